# baseline (speedup 1.0000x reference)
import functools

import jax
import jax.numpy as jnp
from jax import lax
from jax.experimental import pallas as pl
from jax.experimental.pallas import tpu as pltpu
from jax.experimental.pallas import tpu_sc as plsc

NC = 2
NS = 16
NW = NC * NS
KB = 128

_MESH = plsc.VectorSubcoreMesh(
    core_axis_name="c", subcore_axis_name="s", num_cores=NC, num_subcores=NS
)


def _make_deg_kernel(n_pad, nchunk, rpt):
  @functools.partial(
      pl.kernel,
      out_type=jax.ShapeDtypeStruct((n_pad, 128), jnp.float32),
      mesh=_MESH,
      compiler_params=pltpu.CompilerParams(use_tc_tiling_on_sc=False),
      scratch_types=[
          pltpu.VMEM((nchunk, KB), jnp.int32),
          pltpu.VMEM((KB, 16), jnp.float32),
          pltpu.VMEM_SHARED((n_pad, 16), jnp.float32),
          pltpu.SemaphoreType.DMA,
      ],
  )
  def deg_kernel(ei_hbm, ones_hbm, z_hbm, out_hbm, dst_v, ones_v, acc, sem):
    c = lax.axis_index("c")
    s = lax.axis_index("s")
    pltpu.sync_copy(ei_hbm.at[1, c, s], dst_v)
    pltpu.sync_copy(ones_hbm, ones_v)
    pltpu.sync_copy(z_hbm, acc.at[pl.ds(s * rpt, rpt)])
    plsc.subcore_barrier()

    @pl.loop(0, nchunk)
    def _(j):
      pltpu.async_copy(ones_v, acc.at[dst_v.at[j]], sem, add=True)

      @pl.when(j >= 4)
      def _(j=j):
        pltpu.make_async_copy(ones_v, acc.at[dst_v.at[j - 4]], sem).wait()

    for j in range(nchunk - 4, nchunk):
      pltpu.make_async_copy(ones_v, acc.at[dst_v.at[j]], sem).wait()

    plsc.subcore_barrier()
    pltpu.sync_copy(acc.at[pl.ds(s * rpt, rpt)],
                    out_hbm.at[pl.ds(s * rpt, rpt), pl.ds(c * 16, 16)])

  return deg_kernel


def _pipeline(table, src_v, dst_v, rows_v, acc, gsem, ssem, nchunk, slots,
              depth):
  def gather(j, sl):
    pltpu.async_copy(table.at[src_v.at[j]],
                     rows_v.at[pl.ds(sl * KB, KB)], gsem.at[sl])

  def scat_wait(j, sl):
    pltpu.make_async_copy(rows_v.at[pl.ds(sl * KB, KB)],
                          acc.at[dst_v.at[j]], ssem.at[sl]).wait()

  for j in range(depth):
    gather(j, j % slots)

  @pl.loop(0, nchunk)
  def _(j):
    sl = lax.rem(j, slots)
    pltpu.make_async_copy(table.at[src_v.at[j]],
                          rows_v.at[pl.ds(sl * KB, KB)], gsem.at[sl]).wait()
    pltpu.async_copy(rows_v.at[pl.ds(sl * KB, KB)],
                     acc.at[dst_v.at[j]], ssem.at[sl], add=True)

    @pl.when(j + depth >= slots)
    def _():
      scat_wait(j + depth - slots, lax.rem(j + depth - slots, slots))

    @pl.when(j + depth < nchunk)
    def _():
      gather(j + depth, lax.rem(j + depth, slots))

  for j in range(max(0, nchunk - (slots - depth)), nchunk):
    scat_wait(j, j % slots)


def _make_split_agg_kernel(n, feat2, n_pad, nchunk, rpt, slots, depth):
  @functools.partial(
      pl.kernel,
      out_type=jax.ShapeDtypeStruct((n_pad, 128), jnp.float32),
      mesh=_MESH,
      compiler_params=pltpu.CompilerParams(use_tc_tiling_on_sc=False),
      scratch_types=[
          pltpu.VMEM((nchunk, KB), jnp.int32),
          pltpu.VMEM((nchunk, KB), jnp.int32),
          pltpu.VMEM((slots * KB, feat2), jnp.float32),
          pltpu.VMEM_SHARED((n + 8, feat2), jnp.float32),
          pltpu.VMEM_SHARED((n_pad, feat2), jnp.float32),
          pltpu.SemaphoreType.DMA((slots,)),
          pltpu.SemaphoreType.DMA((slots,)),
      ],
  )
  def agg_kernel(full_hbm, ei_hbm, z_hbm, out_hbm,
                 src_v, dst_v, rows_v, table, acc, gsem, ssem):
    c = lax.axis_index("c")
    s = lax.axis_index("s")
    npt = n // NS
    pltpu.sync_copy(ei_hbm.at[0, s], src_v)
    pltpu.sync_copy(ei_hbm.at[1, s], dst_v)
    pltpu.sync_copy(
        full_hbm.at[pl.ds(s * npt, npt), pl.ds(c * feat2, feat2)],
        table.at[pl.ds(s * npt, npt)])
    pltpu.sync_copy(z_hbm, acc.at[pl.ds(s * rpt, rpt)])
    plsc.subcore_barrier()
    _pipeline(table, src_v, dst_v, rows_v, acc, gsem, ssem, nchunk, slots,
              depth)
    plsc.subcore_barrier()
    pltpu.sync_copy(acc.at[pl.ds(s * rpt, rpt)],
                    out_hbm.at[pl.ds(s * rpt, rpt), pl.ds(c * feat2, feat2)])

  return agg_kernel


def _make_agg_kernel(n, feat, n_pad, nchunk, rpt, slots, depth):
  @functools.partial(
      pl.kernel,
      out_type=jax.ShapeDtypeStruct((n_pad, 2 * feat), jnp.float32),
      mesh=_MESH,
      compiler_params=pltpu.CompilerParams(use_tc_tiling_on_sc=False),
      scratch_types=[
          pltpu.VMEM((nchunk, KB), jnp.int32),
          pltpu.VMEM((nchunk, KB), jnp.int32),
          pltpu.VMEM((slots * KB, feat), jnp.float32),
          pltpu.VMEM_SHARED((n + 8, feat), jnp.float32),
          pltpu.VMEM_SHARED((n_pad, feat), jnp.float32),
          pltpu.SemaphoreType.DMA((slots,)),
          pltpu.SemaphoreType.DMA((slots,)),
      ],
  )
  def agg_kernel(table_hbm, ei_hbm, z_hbm, out_hbm,
                 src_v, dst_v, rows_v, table, acc, gsem, ssem):
    c = lax.axis_index("c")
    s = lax.axis_index("s")
    npt = n // NS
    pltpu.sync_copy(ei_hbm.at[0, c, s], src_v)
    pltpu.sync_copy(ei_hbm.at[1, c, s], dst_v)
    pltpu.sync_copy(table_hbm.at[pl.ds(s * npt, npt), pl.ds(0, feat)],
                    table.at[pl.ds(s * npt, npt)])
    pltpu.sync_copy(z_hbm, acc.at[pl.ds(s * rpt, rpt)])
    plsc.subcore_barrier()
    _pipeline(table, src_v, dst_v, rows_v, acc, gsem, ssem, nchunk, slots,
              depth)
    plsc.subcore_barrier()
    pltpu.sync_copy(acc.at[pl.ds(s * rpt, rpt)],
                    out_hbm.at[pl.ds(s * rpt, rpt), pl.ds(c * feat, feat)])

  return agg_kernel


def _dinv(deg_ref):
  deg = deg_ref[:, :1] + deg_ref[:, 16:17] + 2.0
  return lax.rsqrt(deg)


def _tc1_body(x_ref, w_ref, deg_ref, o_ref):
  dinv = _dinv(deg_ref)
  h = jnp.dot(x_ref[...], w_ref[...], preferred_element_type=jnp.float32)
  o_ref[...] = h * dinv


def _tc2_body(agg_ref, hs_ref, deg_ref, b1_ref, g_ref, be_ref, w2_ref, o_ref):
  dinv = _dinv(deg_ref)
  out1 = dinv * agg_ref[...] + (2.0 * dinv) * hs_ref[...] + b1_ref[...]
  mu = jnp.mean(out1, axis=-1, keepdims=True)
  xc = out1 - mu
  var = jnp.mean(xc * xc, axis=-1, keepdims=True)
  hn = xc * lax.rsqrt(var + 1e-5) * g_ref[...] + be_ref[...]
  h = jnp.maximum(hn, 0.0)
  o_ref[...] = jnp.dot(h, w2_ref[...], preferred_element_type=jnp.float32) * dinv


def _tc3_body(agg_ref, hs_ref, deg_ref, b2_ref, o_ref):
  dinv = _dinv(deg_ref)
  nc = o_ref.shape[-1]
  a = agg_ref[:, :nc] + agg_ref[:, nc:]
  o = dinv * a + (2.0 * dinv) * hs_ref[:, :nc] + b2_ref[...]
  m = jnp.max(o, axis=-1, keepdims=True)
  z = o - m
  lse = jnp.log(jnp.sum(jnp.exp(z), axis=-1, keepdims=True))
  o_ref[...] = z - lse


def kernel(x, edge_index, train_mask, W1, b1, gamma, beta, W2, b2):
  del train_mask
  n, d = x.shape
  h = W1.shape[1]
  c = W2.shape[1]
  e = edge_index.shape[1]

  per_w = -(-e // (NW * KB)) * KB
  nchunk = per_w // KB
  e_pad = per_w * NW
  rpt = -(-(-(-(n + 1) // NS)) // 8) * 8
  n_pad = rpt * NS
  pad = e_pad - e
  ei_pad = jnp.pad(edge_index, ((0, 0), (0, pad)), constant_values=n)
  h2 = h // 2
  nchunk2 = e_pad // (NS * KB)
  ei32 = ei_pad.reshape(2, NC, NS, nchunk, KB)
  ei16 = ei_pad.reshape(2, NS, nchunk2, KB)

  ones16 = jnp.ones((KB, 16), jnp.float32)
  z16 = jnp.zeros((rpt, 16), jnp.float32)
  zh2 = jnp.zeros((rpt, h2), jnp.float32)
  zc = jnp.zeros((rpt, c), jnp.float32)

  r = 2000
  grid = (n // r,)
  deg = _make_deg_kernel(n_pad, nchunk, rpt)(ei32, ones16, z16)
  hs1 = pl.pallas_call(
      _tc1_body,
      grid=grid,
      in_specs=[
          pl.BlockSpec((r, d), lambda i: (i, 0)),
          pl.BlockSpec((d, h), lambda i: (0, 0)),
          pl.BlockSpec((r, 128), lambda i: (i, 0)),
      ],
      out_specs=pl.BlockSpec((r, h), lambda i: (i, 0)),
      out_shape=jax.ShapeDtypeStruct((n, h), jnp.float32),
  )(x, W1, deg)

  agg1 = _make_split_agg_kernel(n, h2, n_pad, nchunk2, rpt, 3, 3)(
      hs1, ei16, zh2)

  W2p = jnp.pad(W2, ((0, 0), (0, h - c)))
  hs2 = pl.pallas_call(
      _tc2_body,
      grid=grid,
      in_specs=[
          pl.BlockSpec((r, h), lambda i: (i, 0)),
          pl.BlockSpec((r, h), lambda i: (i, 0)),
          pl.BlockSpec((r, 128), lambda i: (i, 0)),
          pl.BlockSpec((1, h), lambda i: (0, 0)),
          pl.BlockSpec((1, h), lambda i: (0, 0)),
          pl.BlockSpec((1, h), lambda i: (0, 0)),
          pl.BlockSpec((h, h), lambda i: (0, 0)),
      ],
      out_specs=pl.BlockSpec((r, h), lambda i: (i, 0)),
      out_shape=jax.ShapeDtypeStruct((n, h), jnp.float32),
  )(agg1, hs1, deg, b1.reshape(1, h), gamma.reshape(1, h), beta.reshape(1, h), W2p)

  agg2 = _make_agg_kernel(n, c, n_pad, nchunk, rpt, 4, 3)(hs2, ei32, zc)

  out = pl.pallas_call(
      _tc3_body,
      grid=grid,
      in_specs=[
          pl.BlockSpec((r, 2 * c), lambda i: (i, 0)),
          pl.BlockSpec((r, h), lambda i: (i, 0)),
          pl.BlockSpec((r, 128), lambda i: (i, 0)),
          pl.BlockSpec((1, c), lambda i: (0, 0)),
      ],
      out_specs=pl.BlockSpec((r, c), lambda i: (i, 0)),
      out_shape=jax.ShapeDtypeStruct((n, c), jnp.float32),
  )(agg2, hs2, deg, b2.reshape(1, c))

  return out


# speedup vs baseline: 1.0640x; 1.0640x over previous
import functools

import jax
import jax.numpy as jnp
from jax import lax
from jax.experimental import pallas as pl
from jax.experimental.pallas import tpu as pltpu
from jax.experimental.pallas import tpu_sc as plsc

NC = 2
NS = 16
NW = NC * NS
KB = 128

_MESH = plsc.VectorSubcoreMesh(
    core_axis_name="c", subcore_axis_name="s", num_cores=NC, num_subcores=NS
)


def _make_deg_kernel(n_pad, nchunk, rpt):
  @functools.partial(
      pl.kernel,
      out_type=jax.ShapeDtypeStruct((n_pad, 128), jnp.float32),
      mesh=_MESH,
      compiler_params=pltpu.CompilerParams(use_tc_tiling_on_sc=False),
      scratch_types=[
          pltpu.VMEM((nchunk, KB), jnp.int32),
          pltpu.VMEM((KB, 16), jnp.float32),
          pltpu.VMEM_SHARED((n_pad, 16), jnp.float32),
          pltpu.SemaphoreType.DMA,
      ],
  )
  def deg_kernel(ei_hbm, ones_hbm, z_hbm, out_hbm, dst_v, ones_v, acc, sem):
    c = lax.axis_index("c")
    s = lax.axis_index("s")
    pltpu.sync_copy(ei_hbm.at[1, c, s], dst_v)
    pltpu.sync_copy(ones_hbm, ones_v)
    pltpu.sync_copy(z_hbm, acc.at[pl.ds(s * rpt, rpt)])
    plsc.subcore_barrier()

    @pl.loop(0, nchunk)
    def _(j):
      pltpu.async_copy(ones_v, acc.at[dst_v.at[j]], sem, add=True)

      @pl.when(j >= 4)
      def _(j=j):
        pltpu.make_async_copy(ones_v, acc.at[dst_v.at[j - 4]], sem).wait()

    for j in range(nchunk - 4, nchunk):
      pltpu.make_async_copy(ones_v, acc.at[dst_v.at[j]], sem).wait()

    plsc.subcore_barrier()
    pltpu.sync_copy(acc.at[pl.ds(s * rpt, rpt)],
                    out_hbm.at[pl.ds(s * rpt, rpt), pl.ds(c * 16, 16)])

  return deg_kernel


def _pipeline(table, src_v, dst_v, rows_v, acc, gsem, ssem, nchunk, slots,
              depth):
  def gather(j, sl):
    pltpu.async_copy(table.at[src_v.at[j]],
                     rows_v.at[pl.ds(sl * KB, KB)], gsem.at[sl])

  def scat_wait(j, sl):
    pltpu.make_async_copy(rows_v.at[pl.ds(sl * KB, KB)],
                          acc.at[dst_v.at[j]], ssem.at[sl]).wait()

  for j in range(depth):
    gather(j, j % slots)

  @pl.loop(0, nchunk)
  def _(j):
    sl = lax.rem(j, slots)
    pltpu.make_async_copy(table.at[src_v.at[j]],
                          rows_v.at[pl.ds(sl * KB, KB)], gsem.at[sl]).wait()
    pltpu.async_copy(rows_v.at[pl.ds(sl * KB, KB)],
                     acc.at[dst_v.at[j]], ssem.at[sl], add=True)

    @pl.when(j + depth >= slots)
    def _():
      scat_wait(j + depth - slots, lax.rem(j + depth - slots, slots))

    @pl.when(j + depth < nchunk)
    def _():
      gather(j + depth, lax.rem(j + depth, slots))

  for j in range(max(0, nchunk - (slots - depth)), nchunk):
    scat_wait(j, j % slots)


def _make_split_agg_kernel(n, feat2, n_pad, nchunk, rpt, slots, depth):
  @functools.partial(
      pl.kernel,
      out_type=jax.ShapeDtypeStruct((n_pad, 128), jnp.float32),
      mesh=_MESH,
      compiler_params=pltpu.CompilerParams(use_tc_tiling_on_sc=False),
      scratch_types=[
          pltpu.VMEM((nchunk, KB), jnp.int32),
          pltpu.VMEM((nchunk, KB), jnp.int32),
          pltpu.VMEM((slots * KB, feat2), jnp.float32),
          pltpu.VMEM_SHARED((n + 8, feat2), jnp.float32),
          pltpu.VMEM_SHARED((n_pad, feat2), jnp.float32),
          pltpu.SemaphoreType.DMA((slots,)),
          pltpu.SemaphoreType.DMA((slots,)),
      ],
  )
  def agg_kernel(full_hbm, ei_hbm, z_hbm, out_hbm,
                 src_v, dst_v, rows_v, table, acc, gsem, ssem):
    c = lax.axis_index("c")
    s = lax.axis_index("s")
    npt = n // NS
    pltpu.sync_copy(ei_hbm.at[0, s], src_v)
    pltpu.sync_copy(ei_hbm.at[1, s], dst_v)
    pltpu.sync_copy(
        full_hbm.at[pl.ds(s * npt, npt), pl.ds(c * feat2, feat2)],
        table.at[pl.ds(s * npt, npt)])
    pltpu.sync_copy(z_hbm, acc.at[pl.ds(s * rpt, rpt)])
    plsc.subcore_barrier()
    _pipeline(table, src_v, dst_v, rows_v, acc, gsem, ssem, nchunk, slots,
              depth)
    plsc.subcore_barrier()
    pltpu.sync_copy(acc.at[pl.ds(s * rpt, rpt)],
                    out_hbm.at[pl.ds(s * rpt, rpt), pl.ds(c * feat2, feat2)])

  return agg_kernel


def _make_agg_kernel(n, feat, n_pad, nchunk, rpt, slots, depth):
  @functools.partial(
      pl.kernel,
      out_type=jax.ShapeDtypeStruct((n_pad, 2 * feat), jnp.float32),
      mesh=_MESH,
      compiler_params=pltpu.CompilerParams(use_tc_tiling_on_sc=False),
      scratch_types=[
          pltpu.VMEM((nchunk, KB), jnp.int32),
          pltpu.VMEM((nchunk, KB), jnp.int32),
          pltpu.VMEM((slots * KB, feat), jnp.float32),
          pltpu.VMEM_SHARED((n + 8, feat), jnp.float32),
          pltpu.VMEM_SHARED((n_pad, feat), jnp.float32),
          pltpu.SemaphoreType.DMA((slots,)),
          pltpu.SemaphoreType.DMA((slots,)),
      ],
  )
  def agg_kernel(table_hbm, ei_hbm, z_hbm, out_hbm,
                 src_v, dst_v, rows_v, table, acc, gsem, ssem):
    c = lax.axis_index("c")
    s = lax.axis_index("s")
    npt = n // NS
    pltpu.sync_copy(ei_hbm.at[0, c, s], src_v)
    pltpu.sync_copy(ei_hbm.at[1, c, s], dst_v)
    pltpu.sync_copy(table_hbm.at[pl.ds(s * npt, npt), pl.ds(0, feat)],
                    table.at[pl.ds(s * npt, npt)])
    pltpu.sync_copy(z_hbm, acc.at[pl.ds(s * rpt, rpt)])
    plsc.subcore_barrier()
    _pipeline(table, src_v, dst_v, rows_v, acc, gsem, ssem, nchunk, slots,
              depth)
    plsc.subcore_barrier()
    pltpu.sync_copy(acc.at[pl.ds(s * rpt, rpt)],
                    out_hbm.at[pl.ds(s * rpt, rpt), pl.ds(c * feat, feat)])

  return agg_kernel


def _dinv(deg_ref):
  deg = deg_ref[:, :1] + deg_ref[:, 16:17] + 2.0
  return lax.rsqrt(deg)


def _tc1_body(x_ref, w_ref, deg_ref, o_ref):
  dinv = _dinv(deg_ref)
  h = jnp.dot(x_ref[...], w_ref[...], preferred_element_type=jnp.float32)
  o_ref[...] = h * dinv


def _tc2_body(agg_ref, hs_ref, deg_ref, b1_ref, g_ref, be_ref, w2_ref, o_ref):
  dinv = _dinv(deg_ref)
  out1 = dinv * agg_ref[...] + (2.0 * dinv) * hs_ref[...] + b1_ref[...]
  mu = jnp.mean(out1, axis=-1, keepdims=True)
  xc = out1 - mu
  var = jnp.mean(xc * xc, axis=-1, keepdims=True)
  hn = xc * lax.rsqrt(var + 1e-5) * g_ref[...] + be_ref[...]
  h = jnp.maximum(hn, 0.0)
  o_ref[...] = jnp.dot(h, w2_ref[...], preferred_element_type=jnp.float32) * dinv


def _tc3_body(agg_ref, hs_ref, deg_ref, b2_ref, o_ref):
  dinv = _dinv(deg_ref)
  nc = o_ref.shape[-1]
  a = agg_ref[:, :nc] + agg_ref[:, nc:]
  o = dinv * a + (2.0 * dinv) * hs_ref[:, :nc] + b2_ref[...]
  m = jnp.max(o, axis=-1, keepdims=True)
  z = o - m
  lse = jnp.log(jnp.sum(jnp.exp(z), axis=-1, keepdims=True))
  o_ref[...] = z - lse


def kernel(x, edge_index, train_mask, W1, b1, gamma, beta, W2, b2):
  del train_mask
  n, d = x.shape
  h = W1.shape[1]
  c = W2.shape[1]
  e = edge_index.shape[1]

  per_w = -(-e // (NW * KB)) * KB
  nchunk = per_w // KB
  e_pad = per_w * NW
  rpt = -(-(-(-(n + 1) // NS)) // 8) * 8
  n_pad = rpt * NS
  pad = e_pad - e
  ei_pad = jnp.pad(edge_index, ((0, 0), (0, pad)), constant_values=n)
  h2 = h // 2
  nchunk2 = e_pad // (NS * KB)
  ei32 = ei_pad.reshape(2, NC, NS, nchunk, KB)
  ei16 = ei_pad.reshape(2, NS, nchunk2, KB)

  ones16 = jnp.ones((KB, 16), jnp.float32)
  z16 = jnp.zeros((rpt, 16), jnp.float32)
  zh2 = jnp.zeros((rpt, h2), jnp.float32)
  zc = jnp.zeros((rpt, c), jnp.float32)

  r = 2000
  grid = (n // r,)
  deg = _make_deg_kernel(n_pad, nchunk, rpt)(ei32, ones16, z16)
  hs1 = pl.pallas_call(
      _tc1_body,
      grid=grid,
      in_specs=[
          pl.BlockSpec((r, d), lambda i: (i, 0)),
          pl.BlockSpec((d, h), lambda i: (0, 0)),
          pl.BlockSpec((r, 128), lambda i: (i, 0)),
      ],
      out_specs=pl.BlockSpec((r, h), lambda i: (i, 0)),
      out_shape=jax.ShapeDtypeStruct((n, h), jnp.float32),
  )(x, W1, deg)

  agg1 = _make_split_agg_kernel(n, h2, n_pad, nchunk2, rpt, 3, 2)(
      hs1, ei16, zh2)

  W2p = jnp.pad(W2, ((0, 0), (0, h - c)))
  hs2 = pl.pallas_call(
      _tc2_body,
      grid=grid,
      in_specs=[
          pl.BlockSpec((r, h), lambda i: (i, 0)),
          pl.BlockSpec((r, h), lambda i: (i, 0)),
          pl.BlockSpec((r, 128), lambda i: (i, 0)),
          pl.BlockSpec((1, h), lambda i: (0, 0)),
          pl.BlockSpec((1, h), lambda i: (0, 0)),
          pl.BlockSpec((1, h), lambda i: (0, 0)),
          pl.BlockSpec((h, h), lambda i: (0, 0)),
      ],
      out_specs=pl.BlockSpec((r, h), lambda i: (i, 0)),
      out_shape=jax.ShapeDtypeStruct((n, h), jnp.float32),
  )(agg1, hs1, deg, b1.reshape(1, h), gamma.reshape(1, h), beta.reshape(1, h), W2p)

  agg2 = _make_agg_kernel(n, c, n_pad, nchunk, rpt, 4, 3)(hs2, ei32, zc)

  out = pl.pallas_call(
      _tc3_body,
      grid=grid,
      in_specs=[
          pl.BlockSpec((r, 2 * c), lambda i: (i, 0)),
          pl.BlockSpec((r, h), lambda i: (i, 0)),
          pl.BlockSpec((r, 128), lambda i: (i, 0)),
          pl.BlockSpec((1, c), lambda i: (0, 0)),
      ],
      out_specs=pl.BlockSpec((r, c), lambda i: (i, 0)),
      out_shape=jax.ShapeDtypeStruct((n, c), jnp.float32),
  )(agg2, hs2, deg, b2.reshape(1, c))

  return out
